# Initial kernel scaffold; baseline (speedup 1.0000x reference)
import jax
import jax.numpy as jnp
from jax.experimental import pallas as pl


def kernel(gt_labels_list, gt_boxes_list, label_embed):
    raise NotImplementedError("write your pallas kernel here")


# speedup vs baseline: 2.5979x; 2.5979x over previous
import functools

import jax
import jax.numpy as jnp
import numpy as np
from jax import lax
from jax.experimental import pallas as pl
from jax.experimental.pallas import tpu as pltpu
from jax.experimental.pallas import tpu_sc as plsc

_B = 16
_G = 100
_NUM_CLASSES = 80
_EMBED = 256
_N = 2 * _B * _G
_PAD = 200
_TGT = _PAD + 900
_NW = 32
_BPW = 128
_WACT = _N // _BPW


def _build_consts():
    with jax.default_device(jax.devices("cpu")[0]):
        nk = jax.random.key(1234)
        nk1, nk2, nk3, nk4 = jax.random.split(nk, 4)
        p = np.asarray(jax.random.uniform(nk1, (_N,)))
        new_label = np.asarray(jax.random.randint(nk2, (_N,), 0, _NUM_CLASSES))
        rand_sign_i = np.asarray(jax.random.randint(nk3, (_N, 4), 0, 2))
        rand_part = np.asarray(jax.random.uniform(nk4, (_N, 4)))
    chosen = p < 0.25
    neg_mask = ((np.arange(_N) // (_B * _G)) % 2 == 1).astype(np.float32)
    rand_sign = rand_sign_i.astype(np.float32) * 2.0 - 1.0
    noise = (rand_part + neg_mask[:, None]) * rand_sign

    r = np.arange(_N)
    b, q = r // _PAD, r % _PAD
    i_src = np.where(q < _G, b * _G + q, _B * _G + b * _G + (q - _G))

    c_w = np.where(chosen, new_label, -1).astype(np.int32)[i_src].reshape(
        _WACT, _BPW)

    n_out = np.ascontiguousarray(noise[i_src].T.astype(np.float32))
    return c_w, n_out


_C_W, _N_OUT = _build_consts()


def _sc_body(labels_hbm, cmap_hbm, table_hbm, out_hbm, lab_v, c_v, rows_v, sem):
    w = lax.axis_index("s") * 2 + lax.axis_index("c")

    @pl.when(w < _WACT)
    def _():
        pltpu.sync_copy(labels_hbm.at[w], lab_v)
        pltpu.sync_copy(cmap_hbm.at[w], c_v)
        for i in range(_BPW // 16):
            sl = pl.ds(i * 16, 16)
            c = c_v[sl]
            lab_v[sl] = jnp.where(c >= 0, c, lab_v[sl])
        pltpu.async_copy(table_hbm.at[lab_v], rows_v, sem).wait()
        pltpu.sync_copy(rows_v, out_hbm.at[pl.ds(w * _BPW, _BPW)])


@functools.cache
def _get_sc_gather():
    return functools.partial(
        pl.kernel,
        mesh=plsc.VectorSubcoreMesh(core_axis_name="c", subcore_axis_name="s"),
        out_type=jax.ShapeDtypeStruct((_N, _EMBED), jnp.float32),
        scratch_types=[
            pltpu.VMEM((_BPW,), jnp.int32),
            pltpu.VMEM((_BPW,), jnp.int32),
            pltpu.VMEM((_BPW, _EMBED), jnp.float32),
            pltpu.SemaphoreType.DMA,
        ],
    )(_sc_body)


def _tc_body(p_ref, s_ref, n_ref, bbox_ref, mask_ref):
    P = p_ref[...]
    S = s_ref[...]
    half = S * 0.5
    base = jnp.concatenate([P[0:2] - half[0:2], P[2:4] + half[2:4]], axis=0)
    xyxy = jnp.clip(base + n_ref[...] * half, 0.0, 1.0)
    cxcy = (xyxy[0:2] + xyxy[2:4]) / 2.0
    nwh = xyxy[2:4] - xyxy[0:2]
    r4 = jnp.clip(jnp.concatenate([cxcy, nwh], axis=0), 1e-3, 1.0 - 1e-3)
    bbox_ref[...] = jnp.log(r4 / (1.0 - r4))
    row = lax.broadcasted_iota(jnp.int32, (_TGT, _TGT), 0)
    col = lax.broadcasted_iota(jnp.int32, (_TGT, _TGT), 1)
    mask_ref[...] = jnp.logical_and(row >= _PAD, col < _PAD)


def _tc_call(p4, s4, n4):
    return pl.pallas_call(
        _tc_body,
        out_shape=(
            jax.ShapeDtypeStruct((4, _N), jnp.float32),
            jax.ShapeDtypeStruct((_TGT, _TGT), jnp.bool_),
        ),
    )(p4, s4, n4)


def kernel(gt_labels_list, gt_boxes_list, label_embed):
    labels2 = jnp.concatenate([gt_labels_list, gt_labels_list], axis=1)
    labels_w = labels2.reshape(_WACT, _BPW).astype(jnp.int32)
    out_label = _get_sc_gather()(labels_w, jnp.asarray(_C_W), label_embed)

    def dup(x):
        return jnp.concatenate([x, x], axis=1).reshape(-1)

    cx = dup(gt_boxes_list[..., 0])
    cy = dup(gt_boxes_list[..., 1])
    w = dup(gt_boxes_list[..., 2])
    h = dup(gt_boxes_list[..., 3])
    p4 = jnp.stack([cx, cy, cx, cy])
    s4 = jnp.stack([w, h, w, h])
    bbox4, attn_mask = _tc_call(p4, s4, jnp.asarray(_N_OUT))

    input_query_label = out_label.reshape(_B, _PAD, _EMBED)
    input_query_bbox = bbox4.T.reshape(_B, _PAD, 4)
    return input_query_label, input_query_bbox, attn_mask
